# Initial kernel scaffold; baseline (speedup 1.0000x reference)
import jax
import jax.numpy as jnp
from jax.experimental import pallas as pl


def kernel(hidden, classes, emb):
    raise NotImplementedError("write your pallas kernel here")


# speedup vs baseline: 633.8696x; 633.8696x over previous
import jax
import jax.numpy as jnp
from jax import lax
from jax.experimental import pallas as pl
from jax.experimental.pallas import tpu as pltpu
from jax.experimental.pallas import tpu_sc as plsc

B, S, D = 1024, 200, 64
C = 100
SP = 208
NVEC = SP // 16
CPAD = 104
NW = 32
ROWS_PER_W = B // NW


def _body(hid_hbm, cls_hbm, ridx_hbm, emb_hbm, out_hbm,
          emb_v, ridx_v, cls_v, hid_v, sem_h):
    wid = lax.axis_index("s") * 2 + lax.axis_index("c")
    base = wid * ROWS_PER_W
    pltpu.sync_copy(emb_hbm, emb_v)

    def row_step(i, carry):
        b = base + i
        cp_h = pltpu.async_copy(hid_hbm.at[b], hid_v.at[pl.ds(0, S)], sem_h)
        pltpu.sync_copy(ridx_hbm.at[b], ridx_v)
        pltpu.sync_copy(cls_hbm.at[b], cls_v)
        cp_h.wait()

        def chunk_step(v, c2):
            g_vec = plsc.load_gather(ridx_v, [cls_v[pl.ds(v * 16, 16)]])
            s0 = v * 16
            for l in range(16):
                c = g_vec[l]
                for k in range(4):
                    plsc.addupdate(hid_v.at[s0 + l, pl.ds(k * 16, 16)],
                                   emb_v[c, pl.ds(k * 16, 16)])
            return c2

        lax.fori_loop(0, NVEC, chunk_step, 0)
        pltpu.sync_copy(hid_v.at[pl.ds(0, S)], out_hbm.at[b])
        return carry

    lax.fori_loop(0, ROWS_PER_W, row_step, 0)


@jax.jit
def kernel(hidden, classes, emb):
    b, s, d = hidden.shape
    max_classes = emb.shape[0]
    perm_key = jax.random.key(42)
    rand_vals = jax.random.uniform(perm_key, (b, max_classes))
    rand_idx = jnp.argsort(rand_vals, axis=1).astype(jnp.int32)
    ridx_pad = jnp.pad(rand_idx, ((0, 0), (0, CPAD - max_classes)))
    cls_pad = jnp.pad(classes.astype(jnp.int32), ((0, 0), (0, SP - s)))

    mesh = plsc.VectorSubcoreMesh(core_axis_name="c", subcore_axis_name="s",
                                  num_cores=2)
    run = pl.kernel(
        _body,
        out_type=jax.ShapeDtypeStruct((B, S, D), jnp.float32),
        mesh=mesh,
        compiler_params=pltpu.CompilerParams(needs_layout_passes=False),
        scratch_types=[
            pltpu.VMEM((C, D), jnp.float32),
            pltpu.VMEM((CPAD,), jnp.int32),
            pltpu.VMEM((SP,), jnp.int32),
            pltpu.VMEM((SP, D), jnp.float32),
            pltpu.SemaphoreType.DMA,
        ],
    )
    return run(hidden, cls_pad, ridx_pad, emb)


# speedup vs baseline: 811.2086x; 1.2798x over previous
import jax
import jax.numpy as jnp
from jax import lax
from jax.experimental import pallas as pl
from jax.experimental.pallas import tpu as pltpu
from jax.experimental.pallas import tpu_sc as plsc

B, S, D = 1024, 200, 64
C = 100
HD = S * D
NVF = 12
TAIL = S - NVF * 16
NW = 32
NCHUNK = 4
CB = B // NCHUNK


def _make_body(rpw):
    def _body(hid_hbm, cls_hbm, ridx_hbm, emb_hbm, out_hbm,
              emb_v, cls_all, ridx_all, hid0, hid1, hid2, hid3,
              sem_s, sem_in, sem_out):
        wid = lax.axis_index("s") * 2 + lax.axis_index("c")
        base = wid * rpw
        bufs = (hid0, hid1, hid2, hid3)

        cp_e = pltpu.async_copy(emb_hbm, emb_v, sem_s)
        cp_c = pltpu.async_copy(cls_hbm.at[pl.ds(base, rpw)], cls_all, sem_s)
        cp_r = pltpu.async_copy(ridx_hbm.at[pl.ds(base, rpw)], ridx_all, sem_s)

        def issue_in(i, buf):
            pltpu.async_copy(hid_hbm.at[base + i], buf, sem_in)

        def drain_out():
            pltpu.make_async_copy(hid_hbm.at[0], hid0, sem_out).wait()

        issue_in(0, hid0)
        issue_in(1, hid1)
        cp_e.wait()
        cp_c.wait()
        cp_r.wait()

        def lanes(buf, i, start, lane0):
            g = plsc.load_gather(ridx_all,
                                 [jnp.full((16,), i, jnp.int32),
                                  cls_all[i, pl.ds(start, 16)]])
            for l in range(lane0, 16):
                eo = g[l] * D
                so = (start + l) * D
                for k in range(0, D, 16):
                    plsc.addupdate(buf.at[pl.ds(so + k, 16)],
                                   emb_v[pl.ds(eo + k, 16)])

        def compute(i, buf):
            pltpu.make_async_copy(hid_hbm.at[0], hid0, sem_in).wait()

            def _chunk(v, c2):
                lanes(buf, i, v * 16, 0)
                return c2

            lax.fori_loop(0, NVF, _chunk, 0)
            lanes(buf, i, S - 16, 16 - TAIL)
            pltpu.async_copy(buf, out_hbm.at[base + i], sem_out)

        compute(0, hid0)
        issue_in(2, hid2)
        compute(1, hid1)
        issue_in(3, hid3)
        compute(2, hid2)
        drain_out()
        issue_in(4, hid0)
        compute(3, hid3)
        drain_out()
        issue_in(5, hid1)

        def quad(q, carry):
            for j in range(4):
                i = 4 * q + j
                compute(i, bufs[j])
                drain_out()
                issue_in(i + 2, bufs[(j + 2) & 3])
            return carry

        lax.fori_loop(1, rpw // 4 - 1, quad, 0)

        compute(rpw - 4, hid0)
        drain_out()
        issue_in(rpw - 2, hid2)
        compute(rpw - 3, hid1)
        drain_out()
        issue_in(rpw - 1, hid3)
        compute(rpw - 2, hid2)
        drain_out()
        compute(rpw - 1, hid3)
        drain_out()
        drain_out()
        drain_out()

    return _body


@jax.jit
def kernel(hidden, classes, emb):
    b, s, d = hidden.shape
    max_classes = emb.shape[0]
    perm_key = jax.random.key(42)
    rand_vals = jax.random.uniform(perm_key, (b, max_classes))
    rand_idx = jnp.argsort(rand_vals, axis=1).astype(jnp.int32)
    cls32 = classes.astype(jnp.int32)
    hid2 = hidden.reshape(b, s * d)
    emb1 = emb.reshape(-1)

    mesh = plsc.VectorSubcoreMesh(core_axis_name="c", subcore_axis_name="s",
                                  num_cores=2)
    rpw = CB // NW
    run = pl.kernel(
        _make_body(rpw),
        out_type=jax.ShapeDtypeStruct((CB, HD), jnp.float32),
        mesh=mesh,
        compiler_params=pltpu.CompilerParams(needs_layout_passes=False),
        scratch_types=[
            pltpu.VMEM((C * D,), jnp.float32),
            pltpu.VMEM((rpw, S), jnp.int32),
            pltpu.VMEM((rpw, C), jnp.int32),
            pltpu.VMEM((HD,), jnp.float32),
            pltpu.VMEM((HD,), jnp.float32),
            pltpu.VMEM((HD,), jnp.float32),
            pltpu.VMEM((HD,), jnp.float32),
            pltpu.SemaphoreType.DMA,
            pltpu.SemaphoreType.DMA,
            pltpu.SemaphoreType.DMA,
        ],
    )
    outs = [run(hid2[t * CB:(t + 1) * CB],
                cls32[t * CB:(t + 1) * CB],
                rand_idx[t * CB:(t + 1) * CB],
                emb1)
            for t in range(NCHUNK)]
    out2 = jnp.concatenate(outs, axis=0)
    return out2.reshape(b, s, d)


# speedup vs baseline: 859.9684x; 1.0601x over previous
import jax
import jax.numpy as jnp
from jax import lax
from jax.experimental import pallas as pl
from jax.experimental.pallas import tpu as pltpu
from jax.experimental.pallas import tpu_sc as plsc

B, S, D = 1024, 200, 64
C = 100
HD = S * D
NVF = 12
TAIL = S - NVF * 16
NW = 32
NCHUNK = 2
CB = B // NCHUNK


def _make_body(rpw):
    def _body(hid_hbm, cls_hbm, ridx_hbm, emb_hbm, out_hbm,
              emb_v, cls_all, ridx_all, hid0, hid1, hid2, hid3,
              sem_s, sem_in, sem_out):
        wid = lax.axis_index("s") * 2 + lax.axis_index("c")
        base = wid * rpw
        bufs = (hid0, hid1, hid2, hid3)

        cp_e = pltpu.async_copy(emb_hbm, emb_v, sem_s)
        cp_c = pltpu.async_copy(cls_hbm.at[pl.ds(base, rpw)], cls_all, sem_s)
        cp_r = pltpu.async_copy(ridx_hbm.at[pl.ds(base, rpw)], ridx_all, sem_s)

        def issue_in(i, buf):
            pltpu.async_copy(hid_hbm.at[base + i], buf, sem_in)

        def drain_out():
            pltpu.make_async_copy(hid_hbm.at[0], hid0, sem_out).wait()

        issue_in(0, hid0)
        issue_in(1, hid1)
        cp_e.wait()
        cp_c.wait()
        cp_r.wait()

        def lanes(buf, i, start, lane0):
            g = plsc.load_gather(ridx_all,
                                 [jnp.full((16,), i, jnp.int32),
                                  cls_all[i, pl.ds(start, 16)]])
            for l in range(lane0, 16):
                eo = g[l] * D
                so = (start + l) * D
                for k in range(0, D, 16):
                    plsc.addupdate(buf.at[pl.ds(so + k, 16)],
                                   emb_v[pl.ds(eo + k, 16)])

        def compute(i, buf):
            pltpu.make_async_copy(hid_hbm.at[0], hid0, sem_in).wait()

            def _chunk(v, c2):
                lanes(buf, i, v * 16, 0)
                return c2

            lax.fori_loop(0, NVF, _chunk, 0)
            lanes(buf, i, S - 16, 16 - TAIL)
            pltpu.async_copy(buf, out_hbm.at[base + i], sem_out)

        compute(0, hid0)
        issue_in(2, hid2)
        compute(1, hid1)
        issue_in(3, hid3)
        compute(2, hid2)
        drain_out()
        issue_in(4, hid0)
        compute(3, hid3)
        drain_out()
        issue_in(5, hid1)

        def quad(q, carry):
            for j in range(4):
                i = 4 * q + j
                compute(i, bufs[j])
                drain_out()
                issue_in(i + 2, bufs[(j + 2) & 3])
            return carry

        lax.fori_loop(1, rpw // 4 - 1, quad, 0)

        compute(rpw - 4, hid0)
        drain_out()
        issue_in(rpw - 2, hid2)
        compute(rpw - 3, hid1)
        drain_out()
        issue_in(rpw - 1, hid3)
        compute(rpw - 2, hid2)
        drain_out()
        compute(rpw - 1, hid3)
        drain_out()
        drain_out()
        drain_out()

    return _body


@jax.jit
def kernel(hidden, classes, emb):
    b, s, d = hidden.shape
    max_classes = emb.shape[0]
    perm_key = jax.random.key(42)
    rand_vals = jax.random.uniform(perm_key, (b, max_classes))
    rand_idx = jnp.argsort(rand_vals, axis=1).astype(jnp.int32)
    cls32 = classes.astype(jnp.int32)
    hid2 = hidden.reshape(b, s * d)
    emb1 = emb.reshape(-1)

    mesh = plsc.VectorSubcoreMesh(core_axis_name="c", subcore_axis_name="s",
                                  num_cores=2)
    rpw = CB // NW
    run = pl.kernel(
        _make_body(rpw),
        out_type=jax.ShapeDtypeStruct((CB, HD), jnp.float32),
        mesh=mesh,
        compiler_params=pltpu.CompilerParams(needs_layout_passes=False),
        scratch_types=[
            pltpu.VMEM((C * D,), jnp.float32),
            pltpu.VMEM((rpw, S), jnp.int32),
            pltpu.VMEM((rpw, C), jnp.int32),
            pltpu.VMEM((HD,), jnp.float32),
            pltpu.VMEM((HD,), jnp.float32),
            pltpu.VMEM((HD,), jnp.float32),
            pltpu.VMEM((HD,), jnp.float32),
            pltpu.SemaphoreType.DMA,
            pltpu.SemaphoreType.DMA,
            pltpu.SemaphoreType.DMA,
        ],
    )
    outs = [run(hid2[t * CB:(t + 1) * CB],
                cls32[t * CB:(t + 1) * CB],
                rand_idx[t * CB:(t + 1) * CB],
                emb1)
            for t in range(NCHUNK)]
    out2 = jnp.concatenate(outs, axis=0)
    return out2.reshape(b, s, d)
